# baseline (speedup 1.0000x reference)
import functools

import jax
import jax.numpy as jnp
from jax import lax
from jax.experimental import pallas as pl
from jax.experimental.pallas import tpu as pltpu
from jax.experimental.pallas import tpu_sc as plsc

B = 8192
HID = 2048
P = 1024
K = 8
TOPK = 2

BM1 = 1024
BM = 512
NW = 32
RW = B // NW


def _gate_body(h_ref, wg_ref, gate_ref, hb_ref):
    h32 = h_ref[...]
    gate_ref[...] = jax.lax.dot(h32, wg_ref[...], preferred_element_type=jnp.float32)
    hb_ref[...] = h32.astype(jnp.bfloat16)


def _topk_weights_body(gate_hbm, out_hbm, gate_v, w_v):
    c = lax.axis_index("c")
    s = lax.axis_index("s")
    wid = s * 2 + c
    pltpu.sync_copy(gate_hbm.at[wid], gate_v)
    for j in range(RW // 16):
        sl = pl.ds(j * 16, 16)
        vs = [gate_v[k, sl] for k in range(K)]
        v1 = vs[0]
        for k in range(1, K):
            v1 = jnp.maximum(v1, vs[k])
        i1 = jnp.full((16,), K, jnp.int32)
        for k in range(K - 1, -1, -1):
            i1 = jnp.where(vs[k] == v1, k, i1)
        v2 = jnp.full((16,), -3.0e38, jnp.float32)
        for k in range(K):
            vk = jnp.where(i1 == k, -3.0e38, vs[k])
            v2 = jnp.maximum(v2, vk)
        i2 = jnp.full((16,), K, jnp.int32)
        for k in range(K - 1, -1, -1):
            keep = jnp.logical_and(vs[k] == v2, i1 != k)
            i2 = jnp.where(keep, k, i2)
        t = jnp.exp(v2 - v1)
        w1 = 1.0 / (1.0 + t)
        w2 = t / (1.0 + t)
        for k in range(K):
            wk = jnp.where(i1 == k, w1, 0.0) + jnp.where(i2 == k, w2, 0.0)
            w_v[k, sl] = wk
    pltpu.sync_copy(w_v, out_hbm.at[wid])


def _moe_body(hb_ref, w_ref, we_ref, b_ref, out_ref):
    w = w_ref[...]
    hb = hb_ref[...]
    acc = jax.lax.dot(w, b_ref[...], preferred_element_type=jnp.float32)
    for k in range(K):
        yk = jax.lax.dot(
            hb, we_ref[:, k * P:(k + 1) * P], preferred_element_type=jnp.float32
        )
        acc = acc + w[:, k:k + 1] * yk
    out_ref[...] = acc


@jax.jit
def kernel(h, W_exp, b_exp, W_gate):
    Wb = W_exp.astype(jnp.bfloat16)
    b2 = b_exp.reshape(K, P)

    gate, hb = pl.pallas_call(
        _gate_body,
        grid=(B // BM1,),
        in_specs=[
            pl.BlockSpec((BM1, HID), lambda i: (i, 0)),
            pl.BlockSpec((HID, K), lambda i: (0, 0)),
        ],
        out_specs=[
            pl.BlockSpec((BM1, K), lambda i: (i, 0)),
            pl.BlockSpec((BM1, HID), lambda i: (i, 0)),
        ],
        out_shape=[
            jax.ShapeDtypeStruct((B, K), jnp.float32),
            jax.ShapeDtypeStruct((B, HID), jnp.bfloat16),
        ],
        compiler_params=pltpu.CompilerParams(vmem_limit_bytes=50 * 1024 * 1024),
    )(h, W_gate)

    gate3 = jnp.transpose(gate.T.reshape(K, NW, RW), (1, 0, 2))

    topk_kernel = functools.partial(
        pl.kernel,
        mesh=plsc.VectorSubcoreMesh(core_axis_name="c", subcore_axis_name="s"),
        out_type=jax.ShapeDtypeStruct((NW, K, RW), jnp.float32),
        scratch_types=[
            pltpu.VMEM((K, RW), jnp.float32),
            pltpu.VMEM((K, RW), jnp.float32),
        ],
    )(_topk_weights_body)
    w3 = topk_kernel(gate3)

    weights = jnp.transpose(w3, (1, 0, 2)).reshape(K, B).T

    return pl.pallas_call(
        _moe_body,
        grid=(B // BM,),
        in_specs=[
            pl.BlockSpec((BM, HID), lambda i: (i, 0)),
            pl.BlockSpec((BM, K), lambda i: (i, 0)),
            pl.BlockSpec((HID, K * P), lambda i: (0, 0)),
            pl.BlockSpec((K, P), lambda i: (0, 0)),
        ],
        out_specs=pl.BlockSpec((BM, P), lambda i: (i, 0)),
        out_shape=jax.ShapeDtypeStruct((B, P), jnp.float32),
        compiler_params=pltpu.CompilerParams(vmem_limit_bytes=48 * 1024 * 1024),
    )(hb, weights, Wb, b2)


# speedup vs baseline: 1.1219x; 1.1219x over previous
import jax
import jax.numpy as jnp
from jax.experimental import pallas as pl
from jax.experimental.pallas import tpu as pltpu

B = 8192
HID = 2048
P = 1024
K = 8
TOPK = 2

BM = 512


def _moe_body(h_ref, wg_ref, w_ref, b_ref, out_ref):
    h32 = h_ref[...]
    gate = jax.lax.dot(h32, wg_ref[...], preferred_element_type=jnp.float32)

    iota = jax.lax.broadcasted_iota(jnp.int32, gate.shape, 1)
    v1 = jnp.max(gate, axis=1, keepdims=True)
    i1 = jnp.min(jnp.where(gate == v1, iota, K), axis=1, keepdims=True)
    masked = jnp.where(iota == i1, -jnp.inf, gate)
    v2 = jnp.max(masked, axis=1, keepdims=True)
    i2 = jnp.min(jnp.where(masked == v2, iota, K), axis=1, keepdims=True)
    t = jnp.exp(v2 - v1)
    w1 = 1.0 / (1.0 + t)
    w2 = t / (1.0 + t)
    weights = (jnp.where(iota == i1, w1, 0.0)
               + jnp.where(iota == i2, w2, 0.0))

    hb = h32.astype(jnp.bfloat16)
    acc = jax.lax.dot(weights, b_ref[...], preferred_element_type=jnp.float32)
    for k in range(K):
        yk = jax.lax.dot(
            hb, w_ref[:, k * P:(k + 1) * P], preferred_element_type=jnp.float32
        )
        acc = acc + weights[:, k:k + 1] * yk
    out_ref[...] = acc


@jax.jit
def kernel(h, W_exp, b_exp, W_gate):
    Wb = W_exp.astype(jnp.bfloat16)
    b2 = b_exp.reshape(K, P)
    grid = (B // BM,)
    return pl.pallas_call(
        _moe_body,
        grid=grid,
        in_specs=[
            pl.BlockSpec((BM, HID), lambda i: (i, 0)),
            pl.BlockSpec((HID, K), lambda i: (0, 0)),
            pl.BlockSpec((HID, K * P), lambda i: (0, 0)),
            pl.BlockSpec((K, P), lambda i: (0, 0)),
        ],
        out_specs=pl.BlockSpec((BM, P), lambda i: (i, 0)),
        out_shape=jax.ShapeDtypeStruct((B, P), jnp.float32),
        compiler_params=pltpu.CompilerParams(
            vmem_limit_bytes=61 * 1024 * 1024,
        ),
    )(h, W_gate, Wb, b2)
